# Initial kernel scaffold; baseline (speedup 1.0000x reference)
import jax
import jax.numpy as jnp
from jax.experimental import pallas as pl


def kernel(x, table):
    raise NotImplementedError("write your pallas kernel here")


# speedup vs baseline: 1.3063x; 1.3063x over previous
import functools

import jax
import jax.numpy as jnp
from jax import lax
from jax.experimental import pallas as pl
from jax.experimental.pallas import tpu as pltpu
from jax.experimental.pallas import tpu_sc as plsc


def _gather_rows(n_total, n_rows, d, ngrp, g):
    mesh = plsc.VectorSubcoreMesh(core_axis_name="c", subcore_axis_name="s")
    per_w = ngrp * g

    @functools.partial(
        pl.kernel,
        mesh=mesh,
        out_type=jax.ShapeDtypeStruct((n_total, d), jnp.float32),
        scratch_types=[
            pltpu.VMEM((ngrp, g), jnp.int32),
            pltpu.VMEM((g, d), jnp.float32),
            pltpu.SemaphoreType.DMA,
        ],
        compiler_params=pltpu.CompilerParams(use_tc_tiling_on_sc=False),
    )
    def run(idx_hbm, tab_hbm, out_hbm, idx_v, rows_v, sem):
        wid = lax.axis_index("s") * 2 + lax.axis_index("c")
        pltpu.sync_copy(idx_hbm.at[wid], idx_v)
        base = wid * per_w

        def body(j, carry):
            pltpu.async_copy(tab_hbm.at[idx_v.at[j]], rows_v, sem).wait()
            pltpu.sync_copy(rows_v, out_hbm.at[pl.ds(base + j * g, g)])
            return carry

        lax.fori_loop(0, ngrp, body, 0)

    return run


def kernel(x, table):
    b, p = x.shape
    v, d = table.shape
    n = b * p
    nw = 32
    g = 128
    per_w = n // nw
    ngrp = per_w // g

    idx = x.reshape(nw, ngrp, g).astype(jnp.int32)
    out = _gather_rows(n, per_w, d, ngrp, g)(idx, table)
    return out.reshape(b, p, d)


# speedup vs baseline: 1.4944x; 1.1440x over previous
import functools

import jax
import jax.numpy as jnp
from jax import lax
from jax.experimental import pallas as pl
from jax.experimental.pallas import tpu as pltpu
from jax.experimental.pallas import tpu_sc as plsc


def _gather_rows(n_total, d, ngrp, g, k):
    mesh = plsc.VectorSubcoreMesh(core_axis_name="c", subcore_axis_name="s")
    per_w = ngrp * g
    nblk = ngrp // k
    npair = nblk // 2
    blk_rows = k * g

    @functools.partial(
        pl.kernel,
        mesh=mesh,
        out_type=jax.ShapeDtypeStruct((n_total, d), jnp.float32),
        scratch_types=[
            pltpu.VMEM((ngrp, g), jnp.int32),
            pltpu.VMEM((blk_rows, d), jnp.float32),
            pltpu.VMEM((blk_rows, d), jnp.float32),
            pltpu.SemaphoreType.DMA,
            pltpu.SemaphoreType.DMA,
        ],
        compiler_params=pltpu.CompilerParams(use_tc_tiling_on_sc=False),
    )
    def run(idx_hbm, tab_hbm, out_hbm, idx_v, buf_a, buf_b, sem_a, sem_b):
        wid = lax.axis_index("s") * 2 + lax.axis_index("c")
        pltpu.sync_copy(idx_hbm.at[wid], idx_v)
        base = wid * per_w

        def fire(blk, buf, sem):
            for j in range(k):
                pltpu.async_copy(
                    tab_hbm.at[idx_v.at[blk * k + j]],
                    buf.at[pl.ds(j * g, g)],
                    sem,
                )

        def drain(buf, sem):
            pltpu.make_async_copy(tab_hbm.at[pl.ds(0, blk_rows)], buf, sem).wait()

        def store(blk, buf):
            pltpu.sync_copy(buf, out_hbm.at[pl.ds(base + blk * blk_rows, blk_rows)])

        fire(0, buf_a, sem_a)

        def pair(p, carry):
            i0 = 2 * p
            drain(buf_a, sem_a)
            fire(i0 + 1, buf_b, sem_b)
            store(i0, buf_a)
            drain(buf_b, sem_b)

            @pl.when(p < npair - 1)
            def _():
                fire(i0 + 2, buf_a, sem_a)

            store(i0 + 1, buf_b)
            return carry

        lax.fori_loop(0, npair, pair, 0)

    return run


def kernel(x, table):
    b, p = x.shape
    v, d = table.shape
    n = b * p
    nw = 32
    g = 128
    k = 10
    per_w = n // nw
    ngrp = per_w // g

    idx = x.reshape(nw, ngrp, g).astype(jnp.int32)
    out = _gather_rows(n, d, ngrp, g, k)(idx, table)
    return out.reshape(b, p, d)
